# baseline (speedup 1.0000x reference)
import functools

import jax
import jax.numpy as jnp
from jax import lax
from jax.experimental import pallas as pl
from jax.experimental.pallas import tpu as pltpu
from jax.experimental.pallas import tpu_sc as plsc

N_NODES = 10000
N_EDGES = 320000
FDIM = 128
N_GRAPHS = 128
N_LABELS = 10

NTILES = 32
NP = 10240
ROWS_PER_TILE = NP // NTILES
GCH = 4
GCHSZ = ROWS_PER_TILE // GCH
ECHSZ = 128
EPH = 8
TOT_CHUNKS = 2560
E2 = TOT_CHUNKS * ECHSZ
BR = 1024
HB = 512
NHB = NP // HB
SEG_ACC = 136
RR_ROWS = NP // NTILES


def _mesh():
    return plsc.VectorSubcoreMesh(core_axis_name="c", subcore_axis_name="s")


@functools.partial(
    pl.kernel,
    out_type=jax.ShapeDtypeStruct((NP, FDIM), jnp.float32),
    mesh=_mesh(),
    scratch_types=[
        pltpu.VMEM((GCH, GCHSZ), jnp.int32),
        pltpu.VMEM((ROWS_PER_TILE, FDIM), jnp.float32),
        pltpu.SemaphoreType.DMA,
    ],
)
def _gather_rows(emb_hbm, x_hbm, out_hbm, idx_v, rows_v, sem):
    c = lax.axis_index("c")
    s = lax.axis_index("s")
    wid = c * 16 + s
    pltpu.sync_copy(x_hbm.at[wid], idx_v)
    for j in range(GCH):
        pltpu.async_copy(
            emb_hbm.at[idx_v.at[j]], rows_v.at[pl.ds(j * GCHSZ, GCHSZ)], sem
        ).wait()
    pltpu.sync_copy(rows_v, out_hbm.at[pl.ds(wid * ROWS_PER_TILE, ROWS_PER_TILE)])


def _make_edge_agg(c0_chunks, c1_chunks):
    nph0 = c0_chunks // EPH
    nph1 = c1_chunks // EPH

    @functools.partial(
        pl.kernel,
        out_type=jax.ShapeDtypeStruct((2, NP, FDIM), jnp.float32),
        mesh=_mesh(),
        scratch_types=[
            pltpu.VMEM_SHARED((NP, FDIM), jnp.float32),
            pltpu.VMEM((2, EPH, ECHSZ), jnp.int32),
            pltpu.VMEM((2, EPH, ECHSZ), jnp.int32),
            pltpu.VMEM((2, ECHSZ, FDIM), jnp.float32),
            pltpu.SemaphoreType.DMA,
            pltpu.SemaphoreType.DMA,
            pltpu.SemaphoreType.DMA,
            pltpu.SemaphoreType.DMA,
            pltpu.SemaphoreType.DMA,
        ],
    )
    def edge_agg(h_hbm, src_hbm, dst_hbm, z_hbm, out_hbm, agg_sp, src_v,
                 dst_v, rows_v, gsem0, gsem1, ssem0, ssem1, isem):
        c = lax.axis_index("c")
        s = lax.axis_index("s")

        pltpu.sync_copy(z_hbm, rows_v.at[0])
        for k in range(5):
            pltpu.sync_copy(rows_v.at[0],
                            agg_sp.at[pl.ds(s * 640 + k * ECHSZ, ECHSZ)])
        plsc.subcore_barrier()

        nph = jnp.where(c == 0, nph0, nph1)
        t0 = jnp.where(c == 0, s * c0_chunks,
                       jnp.minimum(16 * c0_chunks + s * c1_chunks,
                                   TOT_CHUNKS - EPH))

        gsems = (gsem0, gsem1)
        ssems = (ssem0, ssem1)

        def gather(pb, j, b):
            pltpu.async_copy(h_hbm.at[src_v.at[pb, j]], rows_v.at[b],
                             gsems[b])

        def scat(pb, j, b):
            pltpu.async_copy(rows_v.at[b], agg_sp.at[dst_v.at[pb, j]],
                             ssems[b], add=True)

        def wait_g(b):
            pltpu.make_async_copy(h_hbm.at[src_v.at[0, 0]], rows_v.at[b],
                                  gsems[b]).wait()

        def wait_s(b):
            pltpu.make_async_copy(rows_v.at[b], agg_sp.at[dst_v.at[0, 0]],
                                  ssems[b]).wait()

        def stage_idx(p, pb, wait):
            srccp = pltpu.make_async_copy(
                src_hbm.at[pl.ds(t0 + p * EPH, EPH)], src_v.at[pb], isem)
            dstcp = pltpu.make_async_copy(
                dst_hbm.at[pl.ds(t0 + p * EPH, EPH)], dst_v.at[pb], isem)
            if not wait:
                srccp.start()
                dstcp.start()
            else:
                srccp.wait()
                dstcp.wait()

        @pl.when(nph > 0)
        def _():
            stage_idx(0, 0, False)
            stage_idx(0, 0, True)
            gather(0, 0, 0)

        def phase(p, _):
            pb = p % 2

            @pl.when(p + 1 < nph)
            def _():
                stage_idx(p + 1, (p + 1) % 2, False)

            def group(g, _):
                for b in range(2):
                    j = g * 2 + b
                    wait_g(b)
                    scat(pb, j, b)
                    ob = 1 - b
                    if b == 1:
                        wait_s(ob)
                    else:
                        @pl.when(jnp.logical_or(p >= 1, g >= 1))
                        def _():
                            wait_s(ob)
                    if b == 0:
                        gather(pb, j + 1, ob)
                    else:
                        @pl.when(j + 1 < EPH)
                        def _():
                            gather(pb, j + 1, ob)
                return 0

            lax.fori_loop(0, EPH // 2, group, 0)
            @pl.when(p + 1 < nph)
            def _():
                stage_idx(p + 1, (p + 1) % 2, True)
                gather((p + 1) % 2, 0, 0)
            return 0

        lax.fori_loop(0, nph, phase, 0)

        @pl.when(nph > 0)
        def _():
            wait_s(1)
        plsc.subcore_barrier()

        for k in range(5):
            r0 = s * 640 + k * ECHSZ
            pltpu.sync_copy(agg_sp.at[pl.ds(r0, ECHSZ)], rows_v.at[0])
            pltpu.sync_copy(rows_v.at[0], out_hbm.at[c, pl.ds(r0, ECHSZ)])

    return edge_agg


C0_CHUNKS = 160
C1_CHUNKS = 0
_edge_agg = _make_edge_agg(C0_CHUNKS, C1_CHUNKS)


@functools.partial(
    pl.kernel,
    out_type=jax.ShapeDtypeStruct((NTILES, N_GRAPHS, FDIM), jnp.float32),
    mesh=_mesh(),
    scratch_types=[
        pltpu.VMEM((RR_ROWS, FDIM), jnp.float32),
        pltpu.VMEM((RR_ROWS,), jnp.int32),
        pltpu.VMEM((SEG_ACC, FDIM), jnp.float32),
    ],
)
def _seg_max(h_hbm, batch_hbm, out_hbm, hbuf, bbuf, acc):
    c = lax.axis_index("c")
    s = lax.axis_index("s")
    wid = c * 16 + s

    zero = jnp.zeros((16,), jnp.float32)

    def init(i, _):
        for lg in range(FDIM // 16):
            acc[i, pl.ds(lg * 16, 16)] = zero
        return 0

    lax.fori_loop(0, SEG_ACC, init, 0)

    pltpu.sync_copy(h_hbm.at[pl.ds(wid * RR_ROWS, RR_ROWS)], hbuf)
    pltpu.sync_copy(batch_hbm.at[pl.ds(wid * RR_ROWS, RR_ROWS)], bbuf)

    def body(q, _):
        base = q * 16
        segs = bbuf[pl.ds(base, 16)]
        for t in range(16):
            g = segs[t]
            for lg in range(FDIM // 16):
                col = pl.ds(lg * 16, 16)
                acc[g, col] = jnp.maximum(acc[g, col], hbuf[base + t, col])
        return 0

    lax.fori_loop(0, RR_ROWS // 16, body, 0)

    pltpu.sync_copy(acc.at[pl.ds(0, N_GRAPHS)], out_hbm.at[wid])


def _layer_mm_body(a_ref, h_ref, wr_ref, br_ref, wt_ref, o_ref):
    a = a_ref[0] + a_ref[1]
    o = jnp.dot(a, wr_ref[...], preferred_element_type=jnp.float32)
    o += jnp.dot(h_ref[...], wt_ref[...], preferred_element_type=jnp.float32)
    o_ref[...] = jnp.maximum(o + br_ref[...], 0.0)


def _layer_mm(a, h, w_rel, b_rel, w_root):
    row = pl.BlockSpec((BR, FDIM), lambda i: (i, 0))
    full = pl.BlockSpec((FDIM, FDIM), lambda i: (0, 0))
    return pl.pallas_call(
        _layer_mm_body,
        grid=(NP // BR,),
        in_specs=[pl.BlockSpec((2, BR, FDIM), lambda i: (0, i, 0)),
                  row, full,
                  pl.BlockSpec((1, FDIM), lambda i: (0, 0)), full],
        out_specs=row,
        out_shape=jax.ShapeDtypeStruct((NP, FDIM), jnp.float32),
    )(a, h, w_rel, b_rel.reshape(1, FDIM), w_root)


def _head_body(h_ref, b_ref, mp_ref, w1_ref, b1_ref, w2_ref, b2_ref,
               w3_ref, b3_ref, o_ref, sums, counts):
    i = pl.program_id(0)

    @pl.when(i == 0)
    def _():
        sums[...] = jnp.zeros_like(sums)
        counts[...] = jnp.zeros_like(counts)

    @pl.when(i < NHB)
    def _():
        x = h_ref[...]
        seg = b_ref[0, 0, :]
        gids = lax.broadcasted_iota(jnp.int32, (N_GRAPHS, HB), 0)
        onehot = (seg[None, :] == gids).astype(jnp.float32)
        sums[...] += jnp.dot(onehot, x, preferred_element_type=jnp.float32)
        counts[:, 0:1] += jnp.sum(onehot, axis=1, keepdims=True)

    @pl.when(i == NHB)
    def _():
        maxp = jnp.max(mp_ref[...], axis=0)
        cnt = jnp.maximum(counts[:, 0:1], 1.0)
        meanp = sums[...] / cnt
        g = jnp.concatenate([maxp, meanp], axis=1)
        g = jnp.maximum(jnp.dot(g, w1_ref[...],
                                preferred_element_type=jnp.float32)
                        + b1_ref[...], 0.0)
        g = jnp.maximum(jnp.dot(g, w2_ref[...],
                                preferred_element_type=jnp.float32)
                        + b2_ref[...], 0.0)
        logits = jnp.dot(g, w3_ref[...],
                         preferred_element_type=jnp.float32) + b3_ref[...]
        mx = jnp.max(logits, axis=0, keepdims=True)
        lse = jnp.log(jnp.sum(jnp.exp(logits - mx), axis=0, keepdims=True)) + mx
        o_ref[...] = logits - lse


def _head(h2, batch3, maxp_part, w1, b1, w2, b2, w3p, b3p):
    last = NHB - 1
    return pl.pallas_call(
        _head_body,
        grid=(NHB + 1,),
        in_specs=[
            pl.BlockSpec((HB, FDIM), lambda i: (jnp.minimum(i, last), 0)),
            pl.BlockSpec((1, 1, HB), lambda i: (jnp.minimum(i, last), 0, 0)),
            pl.BlockSpec((NTILES, N_GRAPHS, FDIM), lambda i: (0, 0, 0)),
            pl.BlockSpec((2 * FDIM, FDIM), lambda i: (0, 0)),
            pl.BlockSpec((1, FDIM), lambda i: (0, 0)),
            pl.BlockSpec((FDIM, 64), lambda i: (0, 0)),
            pl.BlockSpec((1, 64), lambda i: (0, 0)),
            pl.BlockSpec((64, 16), lambda i: (0, 0)),
            pl.BlockSpec((1, 16), lambda i: (0, 0)),
        ],
        out_specs=pl.BlockSpec((N_GRAPHS, 16), lambda i: (0, 0)),
        out_shape=jax.ShapeDtypeStruct((N_GRAPHS, 16), jnp.float32),
        scratch_shapes=[
            pltpu.VMEM((N_GRAPHS, FDIM), jnp.float32),
            pltpu.VMEM((N_GRAPHS, 128), jnp.float32),
        ],
    )(h2, batch3, maxp_part, w1, b1.reshape(1, FDIM), w2, b2.reshape(1, 64),
      w3p, b3p)


def kernel(x, edge_index, batch, emb, W_rel1, b_rel1, W_root1,
           W_rel2, b_rel2, W_root2, W1, b1, W2, b2, W3, b3):
    x = x.astype(jnp.int32)
    edge_index = edge_index.astype(jnp.int32)
    batch = batch.astype(jnp.int32)

    x_pad = jnp.pad(x, (0, NP - N_NODES)).reshape(NTILES, GCH, GCHSZ)

    pad_e = E2 - N_EDGES
    src = jnp.pad(edge_index[0], (0, pad_e))
    dst = jnp.concatenate(
        [edge_index[1],
         (N_NODES + (jnp.arange(pad_e, dtype=jnp.int32) % 240))]
    )
    srcb = src.reshape(TOT_CHUNKS, ECHSZ)
    dstb = dst.reshape(TOT_CHUNKS, ECHSZ)

    batch_pad = jnp.pad(batch, (0, NP - N_NODES),
                        constant_values=N_GRAPHS).astype(jnp.int32)
    batch3 = batch_pad.reshape(NHB, 1, HB)

    z = jnp.zeros((ECHSZ, FDIM), jnp.float32)
    w3p = jnp.pad(W3, ((0, 0), (0, 16 - N_LABELS)))
    b3p = jnp.pad(b3, (0, 16 - N_LABELS)).reshape(1, 16)

    h = _gather_rows(emb, x_pad)
    agg1 = _edge_agg(h, srcb, dstb, z)
    h1 = _layer_mm(agg1, h, W_rel1, b_rel1, W_root1)
    agg2 = _edge_agg(h1, srcb, dstb, z)
    h2 = _layer_mm(agg2, h1, W_rel2, b_rel2, W_root2)

    maxp_part = _seg_max(h2, batch_pad)
    out = _head(h2, batch3, maxp_part, W1, b1, W2, b2, w3p, b3p)
    return out[:, :N_LABELS]


# speedup vs baseline: 1.3413x; 1.3413x over previous
import functools

import jax
import jax.numpy as jnp
from jax import lax
from jax.experimental import pallas as pl
from jax.experimental.pallas import tpu as pltpu
from jax.experimental.pallas import tpu_sc as plsc

N_NODES = 10000
N_EDGES = 320000
FDIM = 128
N_GRAPHS = 128
N_LABELS = 10

NTILES = 32
NP = 10240
ROWS_PER_TILE = NP // NTILES
GCH = 4
GCHSZ = ROWS_PER_TILE // GCH
ECHSZ = 128
EPH = 8
TOT_CHUNKS = 2560
E2 = TOT_CHUNKS * ECHSZ
BR = 1024
HB = 512
NHB = NP // HB
SEG_ACC = 136
RR_ROWS = NP // NTILES


def _mesh():
    return plsc.VectorSubcoreMesh(core_axis_name="c", subcore_axis_name="s")


@functools.partial(
    pl.kernel,
    out_type=jax.ShapeDtypeStruct((NP, FDIM), jnp.float32),
    mesh=_mesh(),
    scratch_types=[
        pltpu.VMEM((GCH, GCHSZ), jnp.int32),
        pltpu.VMEM((ROWS_PER_TILE, FDIM), jnp.float32),
        pltpu.SemaphoreType.DMA,
    ],
)
def _gather_rows(emb_hbm, x_hbm, out_hbm, idx_v, rows_v, sem):
    c = lax.axis_index("c")
    s = lax.axis_index("s")
    wid = c * 16 + s
    pltpu.sync_copy(x_hbm.at[wid], idx_v)
    for j in range(GCH):
        pltpu.async_copy(
            emb_hbm.at[idx_v.at[j]], rows_v.at[pl.ds(j * GCHSZ, GCHSZ)], sem
        ).wait()
    pltpu.sync_copy(rows_v, out_hbm.at[pl.ds(wid * ROWS_PER_TILE, ROWS_PER_TILE)])


def _make_edge_agg(c0_chunks, c1_chunks):
    nph0 = c0_chunks // EPH
    nph1 = c1_chunks // EPH

    @functools.partial(
        pl.kernel,
        out_type=jax.ShapeDtypeStruct((2, NP, FDIM), jnp.float32),
        mesh=_mesh(),
        scratch_types=[
            pltpu.VMEM_SHARED((NP, FDIM), jnp.float32),
            pltpu.VMEM((2, EPH, ECHSZ), jnp.int32),
            pltpu.VMEM((2, EPH, ECHSZ), jnp.int32),
            pltpu.VMEM((2, ECHSZ, FDIM), jnp.float32),
            pltpu.SemaphoreType.DMA,
            pltpu.SemaphoreType.DMA,
            pltpu.SemaphoreType.DMA,
            pltpu.SemaphoreType.DMA,
            pltpu.SemaphoreType.DMA,
        ],
    )
    def edge_agg(h_hbm, src_hbm, dst_hbm, z_hbm, out_hbm, agg_sp, src_v,
                 dst_v, rows_v, gsem0, gsem1, ssem0, ssem1, isem):
        c = lax.axis_index("c")
        s = lax.axis_index("s")

        pltpu.sync_copy(z_hbm, rows_v.at[0])
        for k in range(5):
            pltpu.sync_copy(rows_v.at[0],
                            agg_sp.at[pl.ds(s * 640 + k * ECHSZ, ECHSZ)])
        plsc.subcore_barrier()

        nph = jnp.where(c == 0, nph0, nph1)
        t0 = jnp.where(c == 0, s * c0_chunks,
                       jnp.minimum(16 * c0_chunks + s * c1_chunks,
                                   TOT_CHUNKS - EPH))

        gsems = (gsem0, gsem1)
        ssems = (ssem0, ssem1)

        def gather(pb, j, b):
            pltpu.async_copy(h_hbm.at[src_v.at[pb, j]], rows_v.at[b],
                             gsems[b])

        def scat(pb, j, b):
            pltpu.async_copy(rows_v.at[b], agg_sp.at[dst_v.at[pb, j]],
                             ssems[b], add=True)

        def wait_g(b):
            pltpu.make_async_copy(h_hbm.at[src_v.at[0, 0]], rows_v.at[b],
                                  gsems[b]).wait()

        def wait_s(b):
            pltpu.make_async_copy(rows_v.at[b], agg_sp.at[dst_v.at[0, 0]],
                                  ssems[b]).wait()

        def stage_idx(p, pb, wait):
            srccp = pltpu.make_async_copy(
                src_hbm.at[pl.ds(t0 + p * EPH, EPH)], src_v.at[pb], isem)
            dstcp = pltpu.make_async_copy(
                dst_hbm.at[pl.ds(t0 + p * EPH, EPH)], dst_v.at[pb], isem)
            if not wait:
                srccp.start()
                dstcp.start()
            else:
                srccp.wait()
                dstcp.wait()

        @pl.when(nph > 0)
        def _():
            stage_idx(0, 0, False)
            stage_idx(0, 0, True)
            gather(0, 0, 0)

        def phase(p, _):
            pb = p % 2

            @pl.when(p + 1 < nph)
            def _():
                stage_idx(p + 1, (p + 1) % 2, False)

            def group(g, _):
                for b in range(2):
                    j = g * 2 + b
                    wait_g(b)
                    scat(pb, j, b)
                    ob = 1 - b
                    if b == 1:
                        wait_s(ob)
                    else:
                        @pl.when(jnp.logical_or(p >= 1, g >= 1))
                        def _():
                            wait_s(ob)
                    if b == 0:
                        gather(pb, j + 1, ob)
                    else:
                        @pl.when(j + 1 < EPH)
                        def _():
                            gather(pb, j + 1, ob)
                return 0

            lax.fori_loop(0, EPH // 2, group, 0)
            @pl.when(p + 1 < nph)
            def _():
                stage_idx(p + 1, (p + 1) % 2, True)
                gather((p + 1) % 2, 0, 0)
            return 0

        lax.fori_loop(0, nph, phase, 0)

        @pl.when(nph > 0)
        def _():
            wait_s(1)
        plsc.subcore_barrier()

        for k in range(5):
            r0 = s * 640 + k * ECHSZ
            pltpu.sync_copy(agg_sp.at[pl.ds(r0, ECHSZ)], rows_v.at[0])
            pltpu.sync_copy(rows_v.at[0], out_hbm.at[c, pl.ds(r0, ECHSZ)])

    return edge_agg


C0_CHUNKS = 112
C1_CHUNKS = 48
_edge_agg = _make_edge_agg(C0_CHUNKS, C1_CHUNKS)


@functools.partial(
    pl.kernel,
    out_type=jax.ShapeDtypeStruct((NTILES, N_GRAPHS, FDIM), jnp.float32),
    mesh=_mesh(),
    scratch_types=[
        pltpu.VMEM((RR_ROWS, FDIM), jnp.float32),
        pltpu.VMEM((RR_ROWS,), jnp.int32),
        pltpu.VMEM((SEG_ACC, FDIM), jnp.float32),
    ],
)
def _seg_max(h_hbm, batch_hbm, out_hbm, hbuf, bbuf, acc):
    c = lax.axis_index("c")
    s = lax.axis_index("s")
    wid = c * 16 + s

    zero = jnp.zeros((16,), jnp.float32)

    def init(i, _):
        for lg in range(FDIM // 16):
            acc[i, pl.ds(lg * 16, 16)] = zero
        return 0

    lax.fori_loop(0, SEG_ACC, init, 0)

    pltpu.sync_copy(h_hbm.at[pl.ds(wid * RR_ROWS, RR_ROWS)], hbuf)
    pltpu.sync_copy(batch_hbm.at[pl.ds(wid * RR_ROWS, RR_ROWS)], bbuf)

    def body(q, _):
        base = q * 16
        segs = bbuf[pl.ds(base, 16)]
        for t in range(16):
            g = segs[t]
            for lg in range(FDIM // 16):
                col = pl.ds(lg * 16, 16)
                acc[g, col] = jnp.maximum(acc[g, col], hbuf[base + t, col])
        return 0

    lax.fori_loop(0, RR_ROWS // 16, body, 0)

    pltpu.sync_copy(acc.at[pl.ds(0, N_GRAPHS)], out_hbm.at[wid])


def _layer_mm_body(a_ref, h_ref, wr_ref, br_ref, wt_ref, o_ref):
    a = a_ref[0] + a_ref[1]
    o = jnp.dot(a, wr_ref[...], preferred_element_type=jnp.float32)
    o += jnp.dot(h_ref[...], wt_ref[...], preferred_element_type=jnp.float32)
    o_ref[...] = jnp.maximum(o + br_ref[...], 0.0)


def _layer_mm(a, h, w_rel, b_rel, w_root):
    row = pl.BlockSpec((BR, FDIM), lambda i: (i, 0))
    full = pl.BlockSpec((FDIM, FDIM), lambda i: (0, 0))
    return pl.pallas_call(
        _layer_mm_body,
        grid=(NP // BR,),
        in_specs=[pl.BlockSpec((2, BR, FDIM), lambda i: (0, i, 0)),
                  row, full,
                  pl.BlockSpec((1, FDIM), lambda i: (0, 0)), full],
        out_specs=row,
        out_shape=jax.ShapeDtypeStruct((NP, FDIM), jnp.float32),
    )(a, h, w_rel, b_rel.reshape(1, FDIM), w_root)


def _head_body(h_ref, b_ref, mp_ref, w1_ref, b1_ref, w2_ref, b2_ref,
               w3_ref, b3_ref, o_ref, sums, counts):
    i = pl.program_id(0)

    @pl.when(i == 0)
    def _():
        sums[...] = jnp.zeros_like(sums)
        counts[...] = jnp.zeros_like(counts)

    @pl.when(i < NHB)
    def _():
        x = h_ref[...]
        seg = b_ref[0, 0, :]
        gids = lax.broadcasted_iota(jnp.int32, (N_GRAPHS, HB), 0)
        onehot = (seg[None, :] == gids).astype(jnp.float32)
        sums[...] += jnp.dot(onehot, x, preferred_element_type=jnp.float32)
        counts[:, 0:1] += jnp.sum(onehot, axis=1, keepdims=True)

    @pl.when(i == NHB)
    def _():
        maxp = jnp.max(mp_ref[...], axis=0)
        cnt = jnp.maximum(counts[:, 0:1], 1.0)
        meanp = sums[...] / cnt
        g = jnp.concatenate([maxp, meanp], axis=1)
        g = jnp.maximum(jnp.dot(g, w1_ref[...],
                                preferred_element_type=jnp.float32)
                        + b1_ref[...], 0.0)
        g = jnp.maximum(jnp.dot(g, w2_ref[...],
                                preferred_element_type=jnp.float32)
                        + b2_ref[...], 0.0)
        logits = jnp.dot(g, w3_ref[...],
                         preferred_element_type=jnp.float32) + b3_ref[...]
        mx = jnp.max(logits, axis=0, keepdims=True)
        lse = jnp.log(jnp.sum(jnp.exp(logits - mx), axis=0, keepdims=True)) + mx
        o_ref[...] = logits - lse


def _head(h2, batch3, maxp_part, w1, b1, w2, b2, w3p, b3p):
    last = NHB - 1
    return pl.pallas_call(
        _head_body,
        grid=(NHB + 1,),
        in_specs=[
            pl.BlockSpec((HB, FDIM), lambda i: (jnp.minimum(i, last), 0)),
            pl.BlockSpec((1, 1, HB), lambda i: (jnp.minimum(i, last), 0, 0)),
            pl.BlockSpec((NTILES, N_GRAPHS, FDIM), lambda i: (0, 0, 0)),
            pl.BlockSpec((2 * FDIM, FDIM), lambda i: (0, 0)),
            pl.BlockSpec((1, FDIM), lambda i: (0, 0)),
            pl.BlockSpec((FDIM, 64), lambda i: (0, 0)),
            pl.BlockSpec((1, 64), lambda i: (0, 0)),
            pl.BlockSpec((64, 16), lambda i: (0, 0)),
            pl.BlockSpec((1, 16), lambda i: (0, 0)),
        ],
        out_specs=pl.BlockSpec((N_GRAPHS, 16), lambda i: (0, 0)),
        out_shape=jax.ShapeDtypeStruct((N_GRAPHS, 16), jnp.float32),
        scratch_shapes=[
            pltpu.VMEM((N_GRAPHS, FDIM), jnp.float32),
            pltpu.VMEM((N_GRAPHS, 128), jnp.float32),
        ],
    )(h2, batch3, maxp_part, w1, b1.reshape(1, FDIM), w2, b2.reshape(1, 64),
      w3p, b3p)


def kernel(x, edge_index, batch, emb, W_rel1, b_rel1, W_root1,
           W_rel2, b_rel2, W_root2, W1, b1, W2, b2, W3, b3):
    x = x.astype(jnp.int32)
    edge_index = edge_index.astype(jnp.int32)
    batch = batch.astype(jnp.int32)

    x_pad = jnp.pad(x, (0, NP - N_NODES)).reshape(NTILES, GCH, GCHSZ)

    pad_e = E2 - N_EDGES
    src = jnp.pad(edge_index[0], (0, pad_e))
    dst = jnp.concatenate(
        [edge_index[1],
         (N_NODES + (jnp.arange(pad_e, dtype=jnp.int32) % 240))]
    )
    srcb = src.reshape(TOT_CHUNKS, ECHSZ)
    dstb = dst.reshape(TOT_CHUNKS, ECHSZ)

    batch_pad = jnp.pad(batch, (0, NP - N_NODES),
                        constant_values=N_GRAPHS).astype(jnp.int32)
    batch3 = batch_pad.reshape(NHB, 1, HB)

    z = jnp.zeros((ECHSZ, FDIM), jnp.float32)
    w3p = jnp.pad(W3, ((0, 0), (0, 16 - N_LABELS)))
    b3p = jnp.pad(b3, (0, 16 - N_LABELS)).reshape(1, 16)

    h = _gather_rows(emb, x_pad)
    agg1 = _edge_agg(h, srcb, dstb, z)
    h1 = _layer_mm(agg1, h, W_rel1, b_rel1, W_root1)
    agg2 = _edge_agg(h1, srcb, dstb, z)
    h2 = _layer_mm(agg2, h1, W_rel2, b_rel2, W_root2)

    maxp_part = _seg_max(h2, batch_pad)
    out = _head(h2, batch3, maxp_part, W1, b1, W2, b2, w3p, b3p)
    return out[:, :N_LABELS]


# speedup vs baseline: 1.3778x; 1.0272x over previous
import functools

import jax
import jax.numpy as jnp
from jax import lax
from jax.experimental import pallas as pl
from jax.experimental.pallas import tpu as pltpu
from jax.experimental.pallas import tpu_sc as plsc

N_NODES = 10000
N_EDGES = 320000
FDIM = 128
N_GRAPHS = 128
N_LABELS = 10

NTILES = 32
NP = 10240
ROWS_PER_TILE = NP // NTILES
GCH = 4
GCHSZ = ROWS_PER_TILE // GCH
ECHSZ = 128
EPH = 8
TOT_CHUNKS = 2560
E2 = TOT_CHUNKS * ECHSZ
BR = 1024
HB = 512
NHB = NP // HB
SEG_ACC = 136
RR_ROWS = NP // NTILES


def _mesh():
    return plsc.VectorSubcoreMesh(core_axis_name="c", subcore_axis_name="s")


@functools.partial(
    pl.kernel,
    out_type=jax.ShapeDtypeStruct((NP, FDIM), jnp.float32),
    mesh=_mesh(),
    scratch_types=[
        pltpu.VMEM((GCH, GCHSZ), jnp.int32),
        pltpu.VMEM((ROWS_PER_TILE, FDIM), jnp.float32),
        pltpu.SemaphoreType.DMA,
    ],
)
def _gather_rows(emb_hbm, x_hbm, out_hbm, idx_v, rows_v, sem):
    c = lax.axis_index("c")
    s = lax.axis_index("s")
    wid = c * 16 + s
    pltpu.sync_copy(x_hbm.at[wid], idx_v)
    for j in range(GCH):
        pltpu.async_copy(
            emb_hbm.at[idx_v.at[j]], rows_v.at[pl.ds(j * GCHSZ, GCHSZ)], sem
        ).wait()
    pltpu.sync_copy(rows_v, out_hbm.at[pl.ds(wid * ROWS_PER_TILE, ROWS_PER_TILE)])


def _make_edge_agg(c0_chunks, c1_chunks):
    nph0 = c0_chunks // EPH
    nph1 = c1_chunks // EPH

    @functools.partial(
        pl.kernel,
        out_type=jax.ShapeDtypeStruct((2, NP, FDIM), jnp.float32),
        mesh=_mesh(),
        scratch_types=[
            pltpu.VMEM_SHARED((NP, FDIM), jnp.float32),
            pltpu.VMEM((2, EPH, ECHSZ), jnp.int32),
            pltpu.VMEM((2, EPH, ECHSZ), jnp.int32),
            pltpu.VMEM((2, ECHSZ, FDIM), jnp.float32),
            pltpu.SemaphoreType.DMA,
            pltpu.SemaphoreType.DMA,
            pltpu.SemaphoreType.DMA,
            pltpu.SemaphoreType.DMA,
            pltpu.SemaphoreType.DMA,
        ],
    )
    def edge_agg(h_hbm, src_hbm, dst_hbm, z_hbm, out_hbm, agg_sp, src_v,
                 dst_v, rows_v, gsem0, gsem1, ssem0, ssem1, isem):
        c = lax.axis_index("c")
        s = lax.axis_index("s")

        pltpu.sync_copy(z_hbm, rows_v.at[0])
        for k in range(5):
            pltpu.sync_copy(rows_v.at[0],
                            agg_sp.at[pl.ds(s * 640 + k * ECHSZ, ECHSZ)])
        plsc.subcore_barrier()

        nph = jnp.where(c == 0, nph0, nph1)
        t0 = jnp.where(c == 0, s * c0_chunks,
                       jnp.minimum(16 * c0_chunks + s * c1_chunks,
                                   TOT_CHUNKS - EPH))

        gsems = (gsem0, gsem1)
        ssems = (ssem0, ssem1)

        def gather(pb, j, b):
            pltpu.async_copy(h_hbm.at[src_v.at[pb, j]], rows_v.at[b],
                             gsems[b])

        def scat(pb, j, b):
            pltpu.async_copy(rows_v.at[b], agg_sp.at[dst_v.at[pb, j]],
                             ssems[b], add=True)

        def wait_g(b):
            pltpu.make_async_copy(h_hbm.at[src_v.at[0, 0]], rows_v.at[b],
                                  gsems[b]).wait()

        def wait_s(b):
            pltpu.make_async_copy(rows_v.at[b], agg_sp.at[dst_v.at[0, 0]],
                                  ssems[b]).wait()

        def stage_idx(p, pb, wait):
            srccp = pltpu.make_async_copy(
                src_hbm.at[pl.ds(t0 + p * EPH, EPH)], src_v.at[pb], isem)
            dstcp = pltpu.make_async_copy(
                dst_hbm.at[pl.ds(t0 + p * EPH, EPH)], dst_v.at[pb], isem)
            if not wait:
                srccp.start()
                dstcp.start()
            else:
                srccp.wait()
                dstcp.wait()

        @pl.when(nph > 0)
        def _():
            stage_idx(0, 0, False)
            stage_idx(0, 0, True)
            gather(0, 0, 0)

        def phase(p, _):
            pb = p % 2

            @pl.when(p + 1 < nph)
            def _():
                stage_idx(p + 1, (p + 1) % 2, False)

            def group(g, _):
                for b in range(2):
                    j = g * 2 + b
                    wait_g(b)
                    scat(pb, j, b)
                    ob = 1 - b
                    if b == 1:
                        wait_s(ob)
                    else:
                        @pl.when(jnp.logical_or(p >= 1, g >= 1))
                        def _():
                            wait_s(ob)
                    if b == 0:
                        gather(pb, j + 1, ob)
                    else:
                        @pl.when(j + 1 < EPH)
                        def _():
                            gather(pb, j + 1, ob)
                return 0

            lax.fori_loop(0, EPH // 2, group, 0)
            @pl.when(p + 1 < nph)
            def _():
                stage_idx(p + 1, (p + 1) % 2, True)
                gather((p + 1) % 2, 0, 0)
            return 0

        lax.fori_loop(0, nph, phase, 0)

        @pl.when(nph > 0)
        def _():
            wait_s(1)
        plsc.subcore_barrier()

        for k in range(5):
            r0 = s * 640 + k * ECHSZ
            pltpu.sync_copy(agg_sp.at[pl.ds(r0, ECHSZ)], rows_v.at[0])
            pltpu.sync_copy(rows_v.at[0], out_hbm.at[c, pl.ds(r0, ECHSZ)])

    return edge_agg


C0_CHUNKS = 128
C1_CHUNKS = 32
_edge_agg = _make_edge_agg(C0_CHUNKS, C1_CHUNKS)


@functools.partial(
    pl.kernel,
    out_type=jax.ShapeDtypeStruct((NTILES, N_GRAPHS, FDIM), jnp.float32),
    mesh=_mesh(),
    scratch_types=[
        pltpu.VMEM((RR_ROWS, FDIM), jnp.float32),
        pltpu.VMEM((RR_ROWS,), jnp.int32),
        pltpu.VMEM((SEG_ACC, FDIM), jnp.float32),
    ],
)
def _seg_max(h_hbm, batch_hbm, out_hbm, hbuf, bbuf, acc):
    c = lax.axis_index("c")
    s = lax.axis_index("s")
    wid = c * 16 + s

    zero = jnp.zeros((16,), jnp.float32)

    def init(i, _):
        for lg in range(FDIM // 16):
            acc[i, pl.ds(lg * 16, 16)] = zero
        return 0

    lax.fori_loop(0, SEG_ACC, init, 0)

    pltpu.sync_copy(h_hbm.at[pl.ds(wid * RR_ROWS, RR_ROWS)], hbuf)
    pltpu.sync_copy(batch_hbm.at[pl.ds(wid * RR_ROWS, RR_ROWS)], bbuf)

    def body(q, _):
        base = q * 16
        segs = bbuf[pl.ds(base, 16)]
        for t in range(16):
            g = segs[t]
            for lg in range(FDIM // 16):
                col = pl.ds(lg * 16, 16)
                acc[g, col] = jnp.maximum(acc[g, col], hbuf[base + t, col])
        return 0

    lax.fori_loop(0, RR_ROWS // 16, body, 0)

    pltpu.sync_copy(acc.at[pl.ds(0, N_GRAPHS)], out_hbm.at[wid])


def _layer_mm_body(a_ref, h_ref, wr_ref, br_ref, wt_ref, o_ref):
    a = a_ref[0] + a_ref[1]
    o = jnp.dot(a, wr_ref[...], preferred_element_type=jnp.float32)
    o += jnp.dot(h_ref[...], wt_ref[...], preferred_element_type=jnp.float32)
    o_ref[...] = jnp.maximum(o + br_ref[...], 0.0)


def _layer_mm(a, h, w_rel, b_rel, w_root):
    row = pl.BlockSpec((BR, FDIM), lambda i: (i, 0))
    full = pl.BlockSpec((FDIM, FDIM), lambda i: (0, 0))
    return pl.pallas_call(
        _layer_mm_body,
        grid=(NP // BR,),
        in_specs=[pl.BlockSpec((2, BR, FDIM), lambda i: (0, i, 0)),
                  row, full,
                  pl.BlockSpec((1, FDIM), lambda i: (0, 0)), full],
        out_specs=row,
        out_shape=jax.ShapeDtypeStruct((NP, FDIM), jnp.float32),
    )(a, h, w_rel, b_rel.reshape(1, FDIM), w_root)


def _head_body(h_ref, b_ref, mp_ref, w1_ref, b1_ref, w2_ref, b2_ref,
               w3_ref, b3_ref, o_ref, sums, counts):
    i = pl.program_id(0)

    @pl.when(i == 0)
    def _():
        sums[...] = jnp.zeros_like(sums)
        counts[...] = jnp.zeros_like(counts)

    @pl.when(i < NHB)
    def _():
        x = h_ref[...]
        seg = b_ref[0, 0, :]
        gids = lax.broadcasted_iota(jnp.int32, (N_GRAPHS, HB), 0)
        onehot = (seg[None, :] == gids).astype(jnp.float32)
        sums[...] += jnp.dot(onehot, x, preferred_element_type=jnp.float32)
        counts[:, 0:1] += jnp.sum(onehot, axis=1, keepdims=True)

    @pl.when(i == NHB)
    def _():
        maxp = jnp.max(mp_ref[...], axis=0)
        cnt = jnp.maximum(counts[:, 0:1], 1.0)
        meanp = sums[...] / cnt
        g = jnp.concatenate([maxp, meanp], axis=1)
        g = jnp.maximum(jnp.dot(g, w1_ref[...],
                                preferred_element_type=jnp.float32)
                        + b1_ref[...], 0.0)
        g = jnp.maximum(jnp.dot(g, w2_ref[...],
                                preferred_element_type=jnp.float32)
                        + b2_ref[...], 0.0)
        logits = jnp.dot(g, w3_ref[...],
                         preferred_element_type=jnp.float32) + b3_ref[...]
        mx = jnp.max(logits, axis=0, keepdims=True)
        lse = jnp.log(jnp.sum(jnp.exp(logits - mx), axis=0, keepdims=True)) + mx
        o_ref[...] = logits - lse


def _head(h2, batch3, maxp_part, w1, b1, w2, b2, w3p, b3p):
    last = NHB - 1
    return pl.pallas_call(
        _head_body,
        grid=(NHB + 1,),
        in_specs=[
            pl.BlockSpec((HB, FDIM), lambda i: (jnp.minimum(i, last), 0)),
            pl.BlockSpec((1, 1, HB), lambda i: (jnp.minimum(i, last), 0, 0)),
            pl.BlockSpec((NTILES, N_GRAPHS, FDIM), lambda i: (0, 0, 0)),
            pl.BlockSpec((2 * FDIM, FDIM), lambda i: (0, 0)),
            pl.BlockSpec((1, FDIM), lambda i: (0, 0)),
            pl.BlockSpec((FDIM, 64), lambda i: (0, 0)),
            pl.BlockSpec((1, 64), lambda i: (0, 0)),
            pl.BlockSpec((64, 16), lambda i: (0, 0)),
            pl.BlockSpec((1, 16), lambda i: (0, 0)),
        ],
        out_specs=pl.BlockSpec((N_GRAPHS, 16), lambda i: (0, 0)),
        out_shape=jax.ShapeDtypeStruct((N_GRAPHS, 16), jnp.float32),
        scratch_shapes=[
            pltpu.VMEM((N_GRAPHS, FDIM), jnp.float32),
            pltpu.VMEM((N_GRAPHS, 128), jnp.float32),
        ],
    )(h2, batch3, maxp_part, w1, b1.reshape(1, FDIM), w2, b2.reshape(1, 64),
      w3p, b3p)


def kernel(x, edge_index, batch, emb, W_rel1, b_rel1, W_root1,
           W_rel2, b_rel2, W_root2, W1, b1, W2, b2, W3, b3):
    x = x.astype(jnp.int32)
    edge_index = edge_index.astype(jnp.int32)
    batch = batch.astype(jnp.int32)

    x_pad = jnp.pad(x, (0, NP - N_NODES)).reshape(NTILES, GCH, GCHSZ)

    pad_e = E2 - N_EDGES
    src = jnp.pad(edge_index[0], (0, pad_e))
    dst = jnp.concatenate(
        [edge_index[1],
         (N_NODES + (jnp.arange(pad_e, dtype=jnp.int32) % 240))]
    )
    srcb = src.reshape(TOT_CHUNKS, ECHSZ)
    dstb = dst.reshape(TOT_CHUNKS, ECHSZ)

    batch_pad = jnp.pad(batch, (0, NP - N_NODES),
                        constant_values=N_GRAPHS).astype(jnp.int32)
    batch3 = batch_pad.reshape(NHB, 1, HB)

    z = jnp.zeros((ECHSZ, FDIM), jnp.float32)
    w3p = jnp.pad(W3, ((0, 0), (0, 16 - N_LABELS)))
    b3p = jnp.pad(b3, (0, 16 - N_LABELS)).reshape(1, 16)

    h = _gather_rows(emb, x_pad)
    agg1 = _edge_agg(h, srcb, dstb, z)
    h1 = _layer_mm(agg1, h, W_rel1, b_rel1, W_root1)
    agg2 = _edge_agg(h1, srcb, dstb, z)
    h2 = _layer_mm(agg2, h1, W_rel2, b_rel2, W_root2)

    maxp_part = _seg_max(h2, batch_pad)
    out = _head(h2, batch3, maxp_part, W1, b1, W2, b2, w3p, b3p)
    return out[:, :N_LABELS]


# speedup vs baseline: 1.3924x; 1.0106x over previous
import functools

import jax
import jax.numpy as jnp
from jax import lax
from jax.experimental import pallas as pl
from jax.experimental.pallas import tpu as pltpu
from jax.experimental.pallas import tpu_sc as plsc

N_NODES = 10000
N_EDGES = 320000
FDIM = 128
N_GRAPHS = 128
N_LABELS = 10

NTILES = 32
NP = 10240
ROWS_PER_TILE = NP // NTILES
GCH = 4
GCHSZ = ROWS_PER_TILE // GCH
ECHSZ = 128
EPH = 8
TOT_CHUNKS = 2560
E2 = TOT_CHUNKS * ECHSZ
BR = 1024
HB = 512
NHB = NP // HB
SEG_ACC = 136
RR_ROWS = NP // NTILES


def _mesh():
    return plsc.VectorSubcoreMesh(core_axis_name="c", subcore_axis_name="s")


@functools.partial(
    pl.kernel,
    out_type=jax.ShapeDtypeStruct((NP, FDIM), jnp.float32),
    mesh=_mesh(),
    scratch_types=[
        pltpu.VMEM((GCH, GCHSZ), jnp.int32),
        pltpu.VMEM((ROWS_PER_TILE, FDIM), jnp.float32),
        pltpu.SemaphoreType.DMA,
    ],
)
def _gather_rows(emb_hbm, x_hbm, out_hbm, idx_v, rows_v, sem):
    c = lax.axis_index("c")
    s = lax.axis_index("s")
    wid = c * 16 + s
    pltpu.sync_copy(x_hbm.at[wid], idx_v)
    for j in range(GCH):
        pltpu.async_copy(
            emb_hbm.at[idx_v.at[j]], rows_v.at[pl.ds(j * GCHSZ, GCHSZ)], sem
        ).wait()
    pltpu.sync_copy(rows_v, out_hbm.at[pl.ds(wid * ROWS_PER_TILE, ROWS_PER_TILE)])


def _make_edge_agg(c0_chunks, c1_chunks):
    nph0 = c0_chunks // EPH
    nph1 = c1_chunks // EPH

    @functools.partial(
        pl.kernel,
        out_type=jax.ShapeDtypeStruct((2, NP, FDIM), jnp.float32),
        mesh=_mesh(),
        scratch_types=[
            pltpu.VMEM_SHARED((NP, FDIM), jnp.float32),
            pltpu.VMEM((2, EPH, ECHSZ), jnp.int32),
            pltpu.VMEM((2, EPH, ECHSZ), jnp.int32),
            pltpu.VMEM((2, ECHSZ, FDIM), jnp.float32),
            pltpu.SemaphoreType.DMA,
            pltpu.SemaphoreType.DMA,
            pltpu.SemaphoreType.DMA,
            pltpu.SemaphoreType.DMA,
            pltpu.SemaphoreType.DMA,
        ],
    )
    def edge_agg(h_hbm, src_hbm, dst_hbm, z_hbm, out_hbm, agg_sp, src_v,
                 dst_v, rows_v, gsem0, gsem1, ssem0, ssem1, isem):
        c = lax.axis_index("c")
        s = lax.axis_index("s")

        pltpu.sync_copy(z_hbm, rows_v.at[0])
        for k in range(5):
            pltpu.sync_copy(rows_v.at[0],
                            agg_sp.at[pl.ds(s * 640 + k * ECHSZ, ECHSZ)])
        plsc.subcore_barrier()

        nph = jnp.where(c == 0, nph0, nph1)
        t0 = jnp.where(c == 0, s * c0_chunks,
                       jnp.minimum(16 * c0_chunks + s * c1_chunks,
                                   TOT_CHUNKS - EPH))

        gsems = (gsem0, gsem1)
        ssems = (ssem0, ssem1)

        def gather(pb, j, b):
            pltpu.async_copy(h_hbm.at[src_v.at[pb, j]], rows_v.at[b],
                             gsems[b])

        def scat(pb, j, b):
            pltpu.async_copy(rows_v.at[b], agg_sp.at[dst_v.at[pb, j]],
                             ssems[b], add=True)

        def wait_g(b):
            pltpu.make_async_copy(h_hbm.at[src_v.at[0, 0]], rows_v.at[b],
                                  gsems[b]).wait()

        def wait_s(b):
            pltpu.make_async_copy(rows_v.at[b], agg_sp.at[dst_v.at[0, 0]],
                                  ssems[b]).wait()

        def stage_idx(p, pb, wait):
            srccp = pltpu.make_async_copy(
                src_hbm.at[pl.ds(t0 + p * EPH, EPH)], src_v.at[pb], isem)
            dstcp = pltpu.make_async_copy(
                dst_hbm.at[pl.ds(t0 + p * EPH, EPH)], dst_v.at[pb], isem)
            if not wait:
                srccp.start()
                dstcp.start()
            else:
                srccp.wait()
                dstcp.wait()

        @pl.when(nph > 0)
        def _():
            stage_idx(0, 0, False)
            stage_idx(0, 0, True)
            gather(0, 0, 0)

        def phase(p, _):
            pb = p % 2

            @pl.when(p + 1 < nph)
            def _():
                stage_idx(p + 1, (p + 1) % 2, False)

            def group(g, _):
                for b in range(2):
                    j = g * 2 + b
                    wait_g(b)
                    scat(pb, j, b)
                    ob = 1 - b
                    if b == 1:
                        wait_s(ob)
                    else:
                        @pl.when(jnp.logical_or(p >= 1, g >= 1))
                        def _():
                            wait_s(ob)
                    if b == 0:
                        gather(pb, j + 1, ob)
                    else:
                        @pl.when(j + 1 < EPH)
                        def _():
                            gather(pb, j + 1, ob)
                return 0

            lax.fori_loop(0, EPH // 2, group, 0)
            @pl.when(p + 1 < nph)
            def _():
                stage_idx(p + 1, (p + 1) % 2, True)
                gather((p + 1) % 2, 0, 0)
            return 0

        lax.fori_loop(0, nph, phase, 0)

        @pl.when(nph > 0)
        def _():
            wait_s(1)
        plsc.subcore_barrier()

        for k in range(5):
            r0 = s * 640 + k * ECHSZ
            pltpu.sync_copy(agg_sp.at[pl.ds(r0, ECHSZ)], rows_v.at[0])
            pltpu.sync_copy(rows_v.at[0], out_hbm.at[c, pl.ds(r0, ECHSZ)])

    return edge_agg


C0_CHUNKS = 136
C1_CHUNKS = 24
_edge_agg = _make_edge_agg(C0_CHUNKS, C1_CHUNKS)


@functools.partial(
    pl.kernel,
    out_type=jax.ShapeDtypeStruct((NTILES, N_GRAPHS, FDIM), jnp.float32),
    mesh=_mesh(),
    scratch_types=[
        pltpu.VMEM((RR_ROWS, FDIM), jnp.float32),
        pltpu.VMEM((RR_ROWS,), jnp.int32),
        pltpu.VMEM((SEG_ACC, FDIM), jnp.float32),
    ],
)
def _seg_max(h_hbm, batch_hbm, out_hbm, hbuf, bbuf, acc):
    c = lax.axis_index("c")
    s = lax.axis_index("s")
    wid = c * 16 + s

    zero = jnp.zeros((16,), jnp.float32)

    def init(i, _):
        for lg in range(FDIM // 16):
            acc[i, pl.ds(lg * 16, 16)] = zero
        return 0

    lax.fori_loop(0, SEG_ACC, init, 0)

    pltpu.sync_copy(h_hbm.at[pl.ds(wid * RR_ROWS, RR_ROWS)], hbuf)
    pltpu.sync_copy(batch_hbm.at[pl.ds(wid * RR_ROWS, RR_ROWS)], bbuf)

    def body(q, _):
        base = q * 16
        segs = bbuf[pl.ds(base, 16)]
        for t in range(16):
            g = segs[t]
            for lg in range(FDIM // 16):
                col = pl.ds(lg * 16, 16)
                acc[g, col] = jnp.maximum(acc[g, col], hbuf[base + t, col])
        return 0

    lax.fori_loop(0, RR_ROWS // 16, body, 0)

    pltpu.sync_copy(acc.at[pl.ds(0, N_GRAPHS)], out_hbm.at[wid])


def _layer_mm_body(a_ref, h_ref, wr_ref, br_ref, wt_ref, o_ref):
    a = a_ref[0] + a_ref[1]
    o = jnp.dot(a, wr_ref[...], preferred_element_type=jnp.float32)
    o += jnp.dot(h_ref[...], wt_ref[...], preferred_element_type=jnp.float32)
    o_ref[...] = jnp.maximum(o + br_ref[...], 0.0)


def _layer_mm(a, h, w_rel, b_rel, w_root):
    row = pl.BlockSpec((BR, FDIM), lambda i: (i, 0))
    full = pl.BlockSpec((FDIM, FDIM), lambda i: (0, 0))
    return pl.pallas_call(
        _layer_mm_body,
        grid=(NP // BR,),
        in_specs=[pl.BlockSpec((2, BR, FDIM), lambda i: (0, i, 0)),
                  row, full,
                  pl.BlockSpec((1, FDIM), lambda i: (0, 0)), full],
        out_specs=row,
        out_shape=jax.ShapeDtypeStruct((NP, FDIM), jnp.float32),
    )(a, h, w_rel, b_rel.reshape(1, FDIM), w_root)


def _head_body(h_ref, b_ref, mp_ref, w1_ref, b1_ref, w2_ref, b2_ref,
               w3_ref, b3_ref, o_ref, sums, counts):
    i = pl.program_id(0)

    @pl.when(i == 0)
    def _():
        sums[...] = jnp.zeros_like(sums)
        counts[...] = jnp.zeros_like(counts)

    @pl.when(i < NHB)
    def _():
        x = h_ref[...]
        seg = b_ref[0, 0, :]
        gids = lax.broadcasted_iota(jnp.int32, (N_GRAPHS, HB), 0)
        onehot = (seg[None, :] == gids).astype(jnp.float32)
        sums[...] += jnp.dot(onehot, x, preferred_element_type=jnp.float32)
        counts[:, 0:1] += jnp.sum(onehot, axis=1, keepdims=True)

    @pl.when(i == NHB)
    def _():
        maxp = jnp.max(mp_ref[...], axis=0)
        cnt = jnp.maximum(counts[:, 0:1], 1.0)
        meanp = sums[...] / cnt
        g = jnp.concatenate([maxp, meanp], axis=1)
        g = jnp.maximum(jnp.dot(g, w1_ref[...],
                                preferred_element_type=jnp.float32)
                        + b1_ref[...], 0.0)
        g = jnp.maximum(jnp.dot(g, w2_ref[...],
                                preferred_element_type=jnp.float32)
                        + b2_ref[...], 0.0)
        logits = jnp.dot(g, w3_ref[...],
                         preferred_element_type=jnp.float32) + b3_ref[...]
        mx = jnp.max(logits, axis=0, keepdims=True)
        lse = jnp.log(jnp.sum(jnp.exp(logits - mx), axis=0, keepdims=True)) + mx
        o_ref[...] = logits - lse


def _head(h2, batch3, maxp_part, w1, b1, w2, b2, w3p, b3p):
    last = NHB - 1
    return pl.pallas_call(
        _head_body,
        grid=(NHB + 1,),
        in_specs=[
            pl.BlockSpec((HB, FDIM), lambda i: (jnp.minimum(i, last), 0)),
            pl.BlockSpec((1, 1, HB), lambda i: (jnp.minimum(i, last), 0, 0)),
            pl.BlockSpec((NTILES, N_GRAPHS, FDIM), lambda i: (0, 0, 0)),
            pl.BlockSpec((2 * FDIM, FDIM), lambda i: (0, 0)),
            pl.BlockSpec((1, FDIM), lambda i: (0, 0)),
            pl.BlockSpec((FDIM, 64), lambda i: (0, 0)),
            pl.BlockSpec((1, 64), lambda i: (0, 0)),
            pl.BlockSpec((64, 16), lambda i: (0, 0)),
            pl.BlockSpec((1, 16), lambda i: (0, 0)),
        ],
        out_specs=pl.BlockSpec((N_GRAPHS, 16), lambda i: (0, 0)),
        out_shape=jax.ShapeDtypeStruct((N_GRAPHS, 16), jnp.float32),
        scratch_shapes=[
            pltpu.VMEM((N_GRAPHS, FDIM), jnp.float32),
            pltpu.VMEM((N_GRAPHS, 128), jnp.float32),
        ],
    )(h2, batch3, maxp_part, w1, b1.reshape(1, FDIM), w2, b2.reshape(1, 64),
      w3p, b3p)


def kernel(x, edge_index, batch, emb, W_rel1, b_rel1, W_root1,
           W_rel2, b_rel2, W_root2, W1, b1, W2, b2, W3, b3):
    x = x.astype(jnp.int32)
    edge_index = edge_index.astype(jnp.int32)
    batch = batch.astype(jnp.int32)

    x_pad = jnp.pad(x, (0, NP - N_NODES)).reshape(NTILES, GCH, GCHSZ)

    pad_e = E2 - N_EDGES
    src = jnp.pad(edge_index[0], (0, pad_e))
    dst = jnp.concatenate(
        [edge_index[1],
         (N_NODES + (jnp.arange(pad_e, dtype=jnp.int32) % 240))]
    )
    srcb = src.reshape(TOT_CHUNKS, ECHSZ)
    dstb = dst.reshape(TOT_CHUNKS, ECHSZ)

    batch_pad = jnp.pad(batch, (0, NP - N_NODES),
                        constant_values=N_GRAPHS).astype(jnp.int32)
    batch3 = batch_pad.reshape(NHB, 1, HB)

    z = jnp.zeros((ECHSZ, FDIM), jnp.float32)
    w3p = jnp.pad(W3, ((0, 0), (0, 16 - N_LABELS)))
    b3p = jnp.pad(b3, (0, 16 - N_LABELS)).reshape(1, 16)

    h = _gather_rows(emb, x_pad)
    agg1 = _edge_agg(h, srcb, dstb, z)
    h1 = _layer_mm(agg1, h, W_rel1, b_rel1, W_root1)
    agg2 = _edge_agg(h1, srcb, dstb, z)
    h2 = _layer_mm(agg2, h1, W_rel2, b_rel2, W_root2)

    maxp_part = _seg_max(h2, batch_pad)
    out = _head(h2, batch3, maxp_part, W1, b1, W2, b2, w3p, b3p)
    return out[:, :N_LABELS]


# speedup vs baseline: 1.4270x; 1.0248x over previous
import functools

import jax
import jax.numpy as jnp
from jax import lax
from jax.experimental import pallas as pl
from jax.experimental.pallas import tpu as pltpu
from jax.experimental.pallas import tpu_sc as plsc

N_NODES = 10000
N_EDGES = 320000
FDIM = 128
N_GRAPHS = 128
N_LABELS = 10

NTILES = 32
NP = 10240
ROWS_PER_TILE = NP // NTILES
GCH = 4
GCHSZ = ROWS_PER_TILE // GCH
ECHSZ = 128
EPH = 8
TOT_CHUNKS = 2560
E2 = TOT_CHUNKS * ECHSZ
BR = 1024
HB = 512
NHB = NP // HB
SEG_ACC = 136
RR_ROWS = NP // NTILES


def _mesh():
    return plsc.VectorSubcoreMesh(core_axis_name="c", subcore_axis_name="s")


@functools.partial(
    pl.kernel,
    out_type=jax.ShapeDtypeStruct((NP, FDIM), jnp.float32),
    mesh=_mesh(),
    scratch_types=[
        pltpu.VMEM((GCH, GCHSZ), jnp.int32),
        pltpu.VMEM((ROWS_PER_TILE, FDIM), jnp.float32),
        pltpu.SemaphoreType.DMA,
    ],
)
def _gather_rows(emb_hbm, x_hbm, out_hbm, idx_v, rows_v, sem):
    c = lax.axis_index("c")
    s = lax.axis_index("s")
    wid = c * 16 + s
    pltpu.sync_copy(x_hbm.at[wid], idx_v)
    for j in range(GCH):
        pltpu.async_copy(
            emb_hbm.at[idx_v.at[j]], rows_v.at[pl.ds(j * GCHSZ, GCHSZ)], sem
        ).wait()
    pltpu.sync_copy(rows_v, out_hbm.at[pl.ds(wid * ROWS_PER_TILE, ROWS_PER_TILE)])


def _make_edge_agg(c0_chunks, c1_chunks):
    nph0 = c0_chunks // EPH
    nph1 = c1_chunks // EPH

    @functools.partial(
        pl.kernel,
        out_type=jax.ShapeDtypeStruct((2, NP, FDIM), jnp.float32),
        mesh=_mesh(),
        scratch_types=[
            pltpu.VMEM_SHARED((NP, FDIM), jnp.float32),
            pltpu.VMEM((2, EPH, ECHSZ), jnp.int32),
            pltpu.VMEM((2, EPH, ECHSZ), jnp.int32),
            pltpu.VMEM((2, ECHSZ, FDIM), jnp.float32),
            pltpu.SemaphoreType.DMA,
            pltpu.SemaphoreType.DMA,
            pltpu.SemaphoreType.DMA,
            pltpu.SemaphoreType.DMA,
            pltpu.SemaphoreType.DMA,
        ],
    )
    def edge_agg(h_hbm, src_hbm, dst_hbm, z_hbm, out_hbm, agg_sp, src_v,
                 dst_v, rows_v, gsem0, gsem1, ssem0, ssem1, isem):
        c = lax.axis_index("c")
        s = lax.axis_index("s")

        pltpu.sync_copy(z_hbm, rows_v.at[0])
        for k in range(5):
            pltpu.sync_copy(rows_v.at[0],
                            agg_sp.at[pl.ds(s * 640 + k * ECHSZ, ECHSZ)])
        plsc.subcore_barrier()

        nph = jnp.where(c == 0, nph0, nph1)
        t0 = jnp.where(c == 0, s * c0_chunks,
                       jnp.minimum(16 * c0_chunks + s * c1_chunks,
                                   TOT_CHUNKS - EPH))

        gsems = (gsem0, gsem1)
        ssems = (ssem0, ssem1)

        def gather(pb, j, b):
            pltpu.async_copy(h_hbm.at[src_v.at[pb, j]], rows_v.at[b],
                             gsems[b])

        def scat(pb, j, b):
            pltpu.async_copy(rows_v.at[b], agg_sp.at[dst_v.at[pb, j]],
                             ssems[b], add=True)

        def wait_g(b):
            pltpu.make_async_copy(h_hbm.at[src_v.at[0, 0]], rows_v.at[b],
                                  gsems[b]).wait()

        def wait_s(b):
            pltpu.make_async_copy(rows_v.at[b], agg_sp.at[dst_v.at[0, 0]],
                                  ssems[b]).wait()

        def stage_idx(p, pb, wait):
            srccp = pltpu.make_async_copy(
                src_hbm.at[pl.ds(t0 + p * EPH, EPH)], src_v.at[pb], isem)
            dstcp = pltpu.make_async_copy(
                dst_hbm.at[pl.ds(t0 + p * EPH, EPH)], dst_v.at[pb], isem)
            if not wait:
                srccp.start()
                dstcp.start()
            else:
                srccp.wait()
                dstcp.wait()

        @pl.when(nph > 0)
        def _():
            stage_idx(0, 0, False)
            stage_idx(0, 0, True)
            gather(0, 0, 0)

        def phase(p, _):
            pb = p % 2

            @pl.when(p + 1 < nph)
            def _():
                stage_idx(p + 1, (p + 1) % 2, False)

            def group(g, _):
                for b in range(2):
                    j = g * 2 + b
                    wait_g(b)
                    scat(pb, j, b)
                    ob = 1 - b
                    if b == 1:
                        wait_s(ob)
                    else:
                        @pl.when(jnp.logical_or(p >= 1, g >= 1))
                        def _():
                            wait_s(ob)
                    if b == 0:
                        gather(pb, j + 1, ob)
                    else:
                        @pl.when(j + 1 < EPH)
                        def _():
                            gather(pb, j + 1, ob)
                return 0

            lax.fori_loop(0, EPH // 2, group, 0)
            @pl.when(p + 1 < nph)
            def _():
                stage_idx(p + 1, (p + 1) % 2, True)
                gather((p + 1) % 2, 0, 0)
            return 0

        lax.fori_loop(0, nph, phase, 0)

        @pl.when(nph > 0)
        def _():
            wait_s(1)
        plsc.subcore_barrier()

        for k in range(5):
            r0 = s * 640 + k * ECHSZ
            pltpu.sync_copy(agg_sp.at[pl.ds(r0, ECHSZ)], rows_v.at[0])
            pltpu.sync_copy(rows_v.at[0], out_hbm.at[c, pl.ds(r0, ECHSZ)])

    return edge_agg


C0_CHUNKS = 144
C1_CHUNKS = 16
_edge_agg = _make_edge_agg(C0_CHUNKS, C1_CHUNKS)


@functools.partial(
    pl.kernel,
    out_type=jax.ShapeDtypeStruct((NTILES, N_GRAPHS, FDIM), jnp.float32),
    mesh=_mesh(),
    scratch_types=[
        pltpu.VMEM((RR_ROWS, FDIM), jnp.float32),
        pltpu.VMEM((RR_ROWS,), jnp.int32),
        pltpu.VMEM((SEG_ACC, FDIM), jnp.float32),
    ],
)
def _seg_max(h_hbm, batch_hbm, out_hbm, hbuf, bbuf, acc):
    c = lax.axis_index("c")
    s = lax.axis_index("s")
    wid = c * 16 + s

    zero = jnp.zeros((16,), jnp.float32)

    def init(i, _):
        for lg in range(FDIM // 16):
            acc[i, pl.ds(lg * 16, 16)] = zero
        return 0

    lax.fori_loop(0, SEG_ACC, init, 0)

    pltpu.sync_copy(h_hbm.at[pl.ds(wid * RR_ROWS, RR_ROWS)], hbuf)
    pltpu.sync_copy(batch_hbm.at[pl.ds(wid * RR_ROWS, RR_ROWS)], bbuf)

    def body(q, _):
        base = q * 16
        segs = bbuf[pl.ds(base, 16)]
        for t in range(16):
            g = segs[t]
            for lg in range(FDIM // 16):
                col = pl.ds(lg * 16, 16)
                acc[g, col] = jnp.maximum(acc[g, col], hbuf[base + t, col])
        return 0

    lax.fori_loop(0, RR_ROWS // 16, body, 0)

    pltpu.sync_copy(acc.at[pl.ds(0, N_GRAPHS)], out_hbm.at[wid])


def _layer_mm_body(a_ref, h_ref, wr_ref, br_ref, wt_ref, o_ref):
    a = a_ref[0] + a_ref[1]
    o = jnp.dot(a, wr_ref[...], preferred_element_type=jnp.float32)
    o += jnp.dot(h_ref[...], wt_ref[...], preferred_element_type=jnp.float32)
    o_ref[...] = jnp.maximum(o + br_ref[...], 0.0)


def _layer_mm(a, h, w_rel, b_rel, w_root):
    row = pl.BlockSpec((BR, FDIM), lambda i: (i, 0))
    full = pl.BlockSpec((FDIM, FDIM), lambda i: (0, 0))
    return pl.pallas_call(
        _layer_mm_body,
        grid=(NP // BR,),
        in_specs=[pl.BlockSpec((2, BR, FDIM), lambda i: (0, i, 0)),
                  row, full,
                  pl.BlockSpec((1, FDIM), lambda i: (0, 0)), full],
        out_specs=row,
        out_shape=jax.ShapeDtypeStruct((NP, FDIM), jnp.float32),
    )(a, h, w_rel, b_rel.reshape(1, FDIM), w_root)


def _head_body(h_ref, b_ref, mp_ref, w1_ref, b1_ref, w2_ref, b2_ref,
               w3_ref, b3_ref, o_ref, sums, counts):
    i = pl.program_id(0)

    @pl.when(i == 0)
    def _():
        sums[...] = jnp.zeros_like(sums)
        counts[...] = jnp.zeros_like(counts)

    @pl.when(i < NHB)
    def _():
        x = h_ref[...]
        seg = b_ref[0, 0, :]
        gids = lax.broadcasted_iota(jnp.int32, (N_GRAPHS, HB), 0)
        onehot = (seg[None, :] == gids).astype(jnp.float32)
        sums[...] += jnp.dot(onehot, x, preferred_element_type=jnp.float32)
        counts[:, 0:1] += jnp.sum(onehot, axis=1, keepdims=True)

    @pl.when(i == NHB)
    def _():
        maxp = jnp.max(mp_ref[...], axis=0)
        cnt = jnp.maximum(counts[:, 0:1], 1.0)
        meanp = sums[...] / cnt
        g = jnp.concatenate([maxp, meanp], axis=1)
        g = jnp.maximum(jnp.dot(g, w1_ref[...],
                                preferred_element_type=jnp.float32)
                        + b1_ref[...], 0.0)
        g = jnp.maximum(jnp.dot(g, w2_ref[...],
                                preferred_element_type=jnp.float32)
                        + b2_ref[...], 0.0)
        logits = jnp.dot(g, w3_ref[...],
                         preferred_element_type=jnp.float32) + b3_ref[...]
        mx = jnp.max(logits, axis=0, keepdims=True)
        lse = jnp.log(jnp.sum(jnp.exp(logits - mx), axis=0, keepdims=True)) + mx
        o_ref[...] = logits - lse


def _head(h2, batch3, maxp_part, w1, b1, w2, b2, w3p, b3p):
    last = NHB - 1
    return pl.pallas_call(
        _head_body,
        grid=(NHB + 1,),
        in_specs=[
            pl.BlockSpec((HB, FDIM), lambda i: (jnp.minimum(i, last), 0)),
            pl.BlockSpec((1, 1, HB), lambda i: (jnp.minimum(i, last), 0, 0)),
            pl.BlockSpec((NTILES, N_GRAPHS, FDIM), lambda i: (0, 0, 0)),
            pl.BlockSpec((2 * FDIM, FDIM), lambda i: (0, 0)),
            pl.BlockSpec((1, FDIM), lambda i: (0, 0)),
            pl.BlockSpec((FDIM, 64), lambda i: (0, 0)),
            pl.BlockSpec((1, 64), lambda i: (0, 0)),
            pl.BlockSpec((64, 16), lambda i: (0, 0)),
            pl.BlockSpec((1, 16), lambda i: (0, 0)),
        ],
        out_specs=pl.BlockSpec((N_GRAPHS, 16), lambda i: (0, 0)),
        out_shape=jax.ShapeDtypeStruct((N_GRAPHS, 16), jnp.float32),
        scratch_shapes=[
            pltpu.VMEM((N_GRAPHS, FDIM), jnp.float32),
            pltpu.VMEM((N_GRAPHS, 128), jnp.float32),
        ],
    )(h2, batch3, maxp_part, w1, b1.reshape(1, FDIM), w2, b2.reshape(1, 64),
      w3p, b3p)


def kernel(x, edge_index, batch, emb, W_rel1, b_rel1, W_root1,
           W_rel2, b_rel2, W_root2, W1, b1, W2, b2, W3, b3):
    x = x.astype(jnp.int32)
    edge_index = edge_index.astype(jnp.int32)
    batch = batch.astype(jnp.int32)

    x_pad = jnp.pad(x, (0, NP - N_NODES)).reshape(NTILES, GCH, GCHSZ)

    pad_e = E2 - N_EDGES
    src = jnp.pad(edge_index[0], (0, pad_e))
    dst = jnp.concatenate(
        [edge_index[1],
         (N_NODES + (jnp.arange(pad_e, dtype=jnp.int32) % 240))]
    )
    srcb = src.reshape(TOT_CHUNKS, ECHSZ)
    dstb = dst.reshape(TOT_CHUNKS, ECHSZ)

    batch_pad = jnp.pad(batch, (0, NP - N_NODES),
                        constant_values=N_GRAPHS).astype(jnp.int32)
    batch3 = batch_pad.reshape(NHB, 1, HB)

    z = jnp.zeros((ECHSZ, FDIM), jnp.float32)
    w3p = jnp.pad(W3, ((0, 0), (0, 16 - N_LABELS)))
    b3p = jnp.pad(b3, (0, 16 - N_LABELS)).reshape(1, 16)

    h = _gather_rows(emb, x_pad)
    agg1 = _edge_agg(h, srcb, dstb, z)
    h1 = _layer_mm(agg1, h, W_rel1, b_rel1, W_root1)
    agg2 = _edge_agg(h1, srcb, dstb, z)
    h2 = _layer_mm(agg2, h1, W_rel2, b_rel2, W_root2)

    maxp_part = _seg_max(h2, batch_pad)
    out = _head(h2, batch3, maxp_part, W1, b1, W2, b2, w3p, b3p)
    return out[:, :N_LABELS]
